# Initial kernel scaffold; baseline (speedup 1.0000x reference)
import jax
import jax.numpy as jnp
from jax.experimental import pallas as pl


def kernel(x, edge_index, edge_weight, W1, b1, W2, b2, W3, b3, W_ih, b_ih, W_hh, b_hh, W_out, b_out):
    raise NotImplementedError("write your pallas kernel here")


# speedup vs baseline: 18.1117x; 18.1117x over previous
import functools

import jax
import jax.numpy as jnp
from jax import lax
from jax.experimental import pallas as pl
from jax.experimental.pallas import tpu as pltpu
from jax.experimental.pallas import tpu_sc as plsc

L, N, D = 8, 10000, 128
E = 320000
HALF = D // 2
NC, NS = 2, 16

_f32 = jnp.float32
_i32 = jnp.int32

_DEG_EPT = E // (NC * NS)
_DEG_CB = 1000
_DEG_NCH = _DEG_EPT // _DEG_CB

_NP = 10240


def _deg_body(dst_hbm, w_hbm, out_hbm, acc, didx, wbuf, zbuf, tout):
  c = lax.axis_index("c")
  t = lax.axis_index("s")
  for i in range(640 // 16):
    zbuf[pl.ds(i * 16, 16)] = jnp.zeros((16,), _f32)

  def per_l(l, carry):
    pltpu.sync_copy(zbuf, acc.at[pl.ds(pl.multiple_of(t * 640, 8), 640)])
    plsc.subcore_barrier()

    def per_chunk(k, carry2):
      eb = pl.multiple_of(l * E + c * (E // NC) + t * _DEG_EPT + k * _DEG_CB, 8)
      pltpu.sync_copy(dst_hbm.at[pl.ds(eb, _DEG_CB)], didx)
      pltpu.sync_copy(w_hbm.at[pl.ds(eb, _DEG_CB)], wbuf)
      pltpu.sync_copy(wbuf, acc.at[didx], add=True)
      return carry2

    lax.fori_loop(0, _DEG_NCH, per_chunk, 0)
    plsc.subcore_barrier()
    ob = pl.multiple_of(c * (L * N) + l * N, 8)

    @pl.when(t < NS - 1)
    def _():
      o = pl.multiple_of(t * 640, 8)
      pltpu.sync_copy(acc.at[pl.ds(o, 640)], tout)
      pltpu.sync_copy(tout, out_hbm.at[pl.ds(ob + o, 640)])

    @pl.when(t == NS - 1)
    def _():
      pltpu.sync_copy(acc.at[pl.ds(9600, 400)], tout.at[pl.ds(0, 400)])
      pltpu.sync_copy(tout.at[pl.ds(0, 400)], out_hbm.at[pl.ds(ob + 9600, 400)])

    return carry

  lax.fori_loop(0, L, per_l, 0)


_deg_call = pl.kernel(
    _deg_body,
    out_type=jax.ShapeDtypeStruct((NC * L * N,), _f32),
    mesh=plsc.VectorSubcoreMesh(core_axis_name="c", subcore_axis_name="s"),
    compiler_params=pltpu.CompilerParams(use_tc_tiling_on_sc=False),
    scratch_types=[
        pltpu.VMEM_SHARED((_NP,), _f32),
        pltpu.VMEM((_DEG_CB,), _i32),
        pltpu.VMEM((_DEG_CB,), _f32),
        pltpu.VMEM((640,), _f32),
        pltpu.VMEM((640,), _f32),
    ],
)

_EPT = E // NS
_CB = 800
_NCH = _EPT // _CB
_RPT = 640
_RLAST = N - (NS - 1) * _RPT


def _conv_body(tpa, tpb, src_hbm, dst_hbm, w_hbm, sa, sb,
               acc, rows_bf, rows, sidx, didx, wbuf, sem):
  c = lax.axis_index("c")
  t = lax.axis_index("s")

  def _stage(nrows, l):
    ra = pl.multiple_of(t * _RPT, 8)
    pltpu.sync_copy(rows.at[pl.ds(0, nrows), :], acc.at[pl.ds(ra, nrows), :])

  def _unstage(nrows, l):
    rb = pl.multiple_of(l * N + t * _RPT, 8)
    ra = pl.multiple_of(t * _RPT, 8)
    pltpu.sync_copy(acc.at[pl.ds(ra, nrows), :], rows.at[pl.ds(0, nrows), :])

    @pl.when(c == 0)
    def _():
      pltpu.sync_copy(rows.at[pl.ds(0, nrows), :], sa.at[pl.ds(rb, nrows), :])

    @pl.when(c == 1)
    def _():
      pltpu.sync_copy(rows.at[pl.ds(0, nrows), :], sb.at[pl.ds(rb, nrows), :])

  def per_l(l, carry):
    def zb(i, carry0):
      for f in range(HALF // 16):
        rows[i, pl.ds(f * 16, 16)] = jnp.zeros((16,), _f32)
      return carry0

    lax.fori_loop(0, _RPT, zb, 0)

    @pl.when(t < NS - 1)
    def _():
      _stage(_RPT, l)

    @pl.when(t == NS - 1)
    def _():
      _stage(_RLAST, l)

    plsc.subcore_barrier()

    def per_chunk(k, carry2):
      eb = pl.multiple_of(l * E + t * _EPT + k * _CB, 8)
      pltpu.sync_copy(src_hbm.at[pl.ds(eb, _CB)], sidx)
      pltpu.sync_copy(dst_hbm.at[pl.ds(eb, _CB)], didx)
      pltpu.sync_copy(w_hbm.at[pl.ds(eb, _CB)], wbuf)

      @pl.when(c == 0)
      def _():
        pltpu.async_copy(tpa.at[sidx], rows_bf, sem).wait()

      @pl.when(c == 1)
      def _():
        pltpu.async_copy(tpb.at[sidx], rows_bf, sem).wait()

      def per_g(g, carry3):
        wv = wbuf[pl.ds(pl.multiple_of(g * 16, 16), 16)]
        for e16 in range(16):
          e = g * 16 + e16
          rows[e, :] = rows_bf[e, :].astype(_f32) * wv[e16]
        return carry3

      lax.fori_loop(0, _CB // 16, per_g, 0)
      pltpu.sync_copy(rows, acc.at[didx], add=True)
      return carry2

    lax.fori_loop(0, _NCH, per_chunk, 0)
    plsc.subcore_barrier()

    @pl.when(t < NS - 1)
    def _():
      _unstage(_RPT, l)

    @pl.when(t == NS - 1)
    def _():
      _unstage(_RLAST, l)

    return carry

  lax.fori_loop(0, L, per_l, 0)


_conv_call = pl.kernel(
    _conv_body,
    out_type=[
        jax.ShapeDtypeStruct((L * N, HALF), _f32),
        jax.ShapeDtypeStruct((L * N, HALF), _f32),
    ],
    mesh=plsc.VectorSubcoreMesh(core_axis_name="c", subcore_axis_name="s"),
    compiler_params=pltpu.CompilerParams(use_tc_tiling_on_sc=False),
    scratch_types=[
        pltpu.VMEM_SHARED((N, HALF), _f32),
        pltpu.VMEM((_CB, HALF), jnp.bfloat16),
        pltpu.VMEM((_CB, HALF), _f32),
        pltpu.VMEM((_CB,), _i32),
        pltpu.VMEM((_CB,), _i32),
        pltpu.VMEM((_CB,), _f32),
        pltpu.SemaphoreType.DMA,
    ],
)

_BN = 2000
_NB = N // _BN


def _prep_block(degp_ref, x_ref, w1_ref, dis_ref, ta_ref, tb_ref,
                tab_ref, tbb_ref):
  deg = degp_ref[0, :, 0] + degp_ref[0, :, 1] + 1.0
  dis = lax.rsqrt(deg)
  dis_ref[0, :, 0] = dis
  tmat = jnp.dot(x_ref[0], w1_ref[...], preferred_element_type=_f32)
  ts = tmat * dis[:, None]
  ta_ref[0] = ts[:, :HALF]
  tb_ref[0] = ts[:, HALF:]
  tab_ref[0] = ts[:, :HALF].astype(jnp.bfloat16)
  tbb_ref[0] = ts[:, HALF:].astype(jnp.bfloat16)


def _prep_call(degp, x, w1):
  return pl.pallas_call(
      _prep_block,
      grid=(L, _NB),
      in_specs=[
          pl.BlockSpec((1, _BN, 2), lambda l, i: (l, i, 0)),
          pl.BlockSpec((1, _BN, D), lambda l, i: (l, i, 0)),
          pl.BlockSpec((D, D), lambda l, i: (0, 0)),
      ],
      out_specs=[
          pl.BlockSpec((1, _BN, 1), lambda l, i: (l, i, 0)),
          pl.BlockSpec((1, _BN, HALF), lambda l, i: (l, i, 0)),
          pl.BlockSpec((1, _BN, HALF), lambda l, i: (l, i, 0)),
          pl.BlockSpec((1, _BN, HALF), lambda l, i: (l, i, 0)),
          pl.BlockSpec((1, _BN, HALF), lambda l, i: (l, i, 0)),
      ],
      out_shape=[
          jax.ShapeDtypeStruct((L, N, 1), _f32),
          jax.ShapeDtypeStruct((L, N, HALF), _f32),
          jax.ShapeDtypeStruct((L, N, HALF), _f32),
          jax.ShapeDtypeStruct((L, N, HALF), jnp.bfloat16),
          jax.ShapeDtypeStruct((L, N, HALF), jnp.bfloat16),
      ],
  )(degp, x, w1)


def _mid_block(sa_ref, sb_ref, ta_ref, tb_ref, dis_ref, b_ref, w_ref,
               ta2_ref, tb2_ref, tab2_ref, tbb2_ref):
  d = dis_ref[0, :, 0]
  s_plus_t = jnp.concatenate(
      [sa_ref[0] + ta_ref[0], sb_ref[0] + tb_ref[0]], axis=1)
  h = jnp.maximum(d[:, None] * s_plus_t + b_ref[0][None, :], 0.0)
  t2 = jnp.dot(h, w_ref[...], preferred_element_type=_f32)
  ts = t2 * d[:, None]
  ta2_ref[0] = ts[:, :HALF]
  tb2_ref[0] = ts[:, HALF:]
  tab2_ref[0] = ts[:, :HALF].astype(jnp.bfloat16)
  tbb2_ref[0] = ts[:, HALF:].astype(jnp.bfloat16)


def _mid_call(sa, sb, ta, tb, dis, b, w):
  return pl.pallas_call(
      _mid_block,
      grid=(L, _NB),
      in_specs=[
          pl.BlockSpec((1, _BN, HALF), lambda l, i: (l, i, 0)),
          pl.BlockSpec((1, _BN, HALF), lambda l, i: (l, i, 0)),
          pl.BlockSpec((1, _BN, HALF), lambda l, i: (l, i, 0)),
          pl.BlockSpec((1, _BN, HALF), lambda l, i: (l, i, 0)),
          pl.BlockSpec((1, _BN, 1), lambda l, i: (l, i, 0)),
          pl.BlockSpec((1, D), lambda l, i: (0, 0)),
          pl.BlockSpec((D, D), lambda l, i: (0, 0)),
      ],
      out_specs=[
          pl.BlockSpec((1, _BN, HALF), lambda l, i: (l, i, 0)),
          pl.BlockSpec((1, _BN, HALF), lambda l, i: (l, i, 0)),
          pl.BlockSpec((1, _BN, HALF), lambda l, i: (l, i, 0)),
          pl.BlockSpec((1, _BN, HALF), lambda l, i: (l, i, 0)),
      ],
      out_shape=[
          jax.ShapeDtypeStruct((L, N, HALF), _f32),
          jax.ShapeDtypeStruct((L, N, HALF), _f32),
          jax.ShapeDtypeStruct((L, N, HALF), jnp.bfloat16),
          jax.ShapeDtypeStruct((L, N, HALF), jnp.bfloat16),
      ],
  )(sa, sb, ta, tb, dis, b, w)


def _fin_block(sa_ref, sb_ref, ta_ref, tb_ref, dis_ref, b_ref, pp_ref):
  d = dis_ref[0, :, 0]
  s_plus_t = jnp.concatenate(
      [sa_ref[0] + ta_ref[0], sb_ref[0] + tb_ref[0]], axis=1)
  z = jnp.maximum(d[:, None] * s_plus_t + b_ref[0][None, :], 0.0)
  pp_ref[0, 0, 0, :] = jnp.sum(z, axis=0)


def _fin_call(sa, sb, ta, tb, dis, b):
  return pl.pallas_call(
      _fin_block,
      grid=(L, _NB),
      in_specs=[
          pl.BlockSpec((1, _BN, HALF), lambda l, i: (l, i, 0)),
          pl.BlockSpec((1, _BN, HALF), lambda l, i: (l, i, 0)),
          pl.BlockSpec((1, _BN, HALF), lambda l, i: (l, i, 0)),
          pl.BlockSpec((1, _BN, HALF), lambda l, i: (l, i, 0)),
          pl.BlockSpec((1, _BN, 1), lambda l, i: (l, i, 0)),
          pl.BlockSpec((1, D), lambda l, i: (0, 0)),
      ],
      out_specs=[pl.BlockSpec((1, 1, 1, D), lambda l, i: (l, i, 0, 0))],
      out_shape=[jax.ShapeDtypeStruct((L, _NB, 1, D), _f32)],
  )(sa, sb, ta, tb, dis, b)[0]


def _rnn_block(pp_ref, wih_t_ref, bih_ref, whh_t_ref, bhh_ref, wout_ref,
               bout_ref, out_ref):
  seq = jnp.sum(pp_ref[...], axis=(1, 2)) * (1.0 / N)
  h = jnp.zeros((1, D), _f32)
  hs = []
  for i in range(L):
    xt = lax.slice(seq, (i, 0), (i + 1, D))
    h = jnp.tanh(
        jnp.dot(xt, wih_t_ref[...], preferred_element_type=_f32)
        + bih_ref[...]
        + jnp.dot(h, whh_t_ref[...], preferred_element_type=_f32)
        + bhh_ref[...])
    hs.append(h)
  hsm = jnp.concatenate(hs, axis=0)
  logits = jnp.dot(hsm, wout_ref[...], preferred_element_type=_f32)
  out_ref[...] = jax.nn.sigmoid(logits + bout_ref[...])


def _rnn_call(pp, wih_t, bih, whh_t, bhh, wout, bout):
  return pl.pallas_call(
      _rnn_block,
      out_shape=jax.ShapeDtypeStruct((L, 2), _f32),
  )(pp, wih_t, bih, whh_t, bhh, wout, bout)


def kernel(x, edge_index, edge_weight, W1, b1, W2, b2, W3, b3,
           W_ih, b_ih, W_hh, b_hh, W_out, b_out):
  loff = (jnp.arange(L, dtype=_i32) * N)[:, None]
  src = (edge_index[:, 0, :] + loff).reshape(L * E)
  dst = edge_index[:, 1, :].reshape(L * E)
  w = edge_weight.reshape(L * E)

  degp = _deg_call(dst, w)
  degt = jnp.transpose(degp.reshape(NC, L, N), (1, 2, 0))

  dis, ta, tb, tab, tbb = _prep_call(degt, x, W1)

  def flat(a):
    return a.reshape(L * N, HALF)

  def unflat(a):
    return a.reshape(L, N, HALF)

  sa, sb = _conv_call(flat(tab), flat(tbb), src, dst, w)
  ta, tb, tab, tbb = _mid_call(unflat(sa), unflat(sb), ta, tb, dis,
                               b1.reshape(1, D), W2)
  sa, sb = _conv_call(flat(tab), flat(tbb), src, dst, w)
  ta, tb, tab, tbb = _mid_call(unflat(sa), unflat(sb), ta, tb, dis,
                               b2.reshape(1, D), W3)
  sa, sb = _conv_call(flat(tab), flat(tbb), src, dst, w)
  pp = _fin_call(unflat(sa), unflat(sb), ta, tb, dis, b3.reshape(1, D))

  return _rnn_call(pp, W_ih.T, b_ih.reshape(1, D), W_hh.T,
                   b_hh.reshape(1, D), W_out, b_out.reshape(1, 2))


# speedup vs baseline: 26.9050x; 1.4855x over previous
import functools

import jax
import jax.numpy as jnp
from jax import lax
from jax.experimental import pallas as pl
from jax.experimental.pallas import tpu as pltpu
from jax.experimental.pallas import tpu_sc as plsc

L, N, D = 8, 10000, 128
E = 320000
HALF = D // 2
NC, NS = 2, 16

_f32 = jnp.float32
_i32 = jnp.int32

_DEG_EPT = E // (NC * NS)
_DEG_CB = 1000
_DEG_NCH = _DEG_EPT // _DEG_CB

_NP = 10240


_CB = 800
_EPT = E // NS
_NCH = _EPT // _CB
_DG = _CB // NC


def _deg_body(epk, w_hbm, out_hbm, acc,
              ebufa, ebufb, wbufa, wbufb, didxa, didxb, wsrca, wsrcb,
              zbuf, tout, isema, isemb, ssema, ssemb):
  c = lax.axis_index("c")
  t = lax.axis_index("s")
  ebuf = (ebufa, ebufb)
  wbuf = (wbufa, wbufb)
  didx = (didxa, didxb)
  wsrc = (wsrca, wsrcb)
  isem = (isema, isemb)
  ssem = (ssema, ssemb)
  for i in range(640 // 16):
    zbuf[pl.ds(i * 16, 16)] = jnp.zeros((16,), _f32)

  def _idx_start(s, l, k):
    row = (l * NS + t) * _NCH + k
    pltpu.async_copy(epk.at[pl.ds(row, 1), :, :], ebuf[s], isem[s])
    eb = pl.multiple_of(l * E + t * _EPT + k * _CB, 8)
    pltpu.async_copy(w_hbm.at[pl.ds(eb, _CB)], wbuf[s], isem[s])

  def _idx_wait(s):
    pltpu.make_async_copy(epk.at[pl.ds(0, 1), :, :], ebuf[s], isem[s]).wait()
    pltpu.make_async_copy(w_hbm.at[pl.ds(0, _CB)], wbuf[s], isem[s]).wait()

  def _scatter_start(s):
    pltpu.async_copy(wsrc[s], acc.at[didx[s]], ssem[s], add=True)

  def _scatter_wait(s):
    pltpu.make_async_copy(wsrc[s], acc.at[didx[s]], ssem[s]).wait()

  def _grab(s):
    def per_g(g, carry3):
      sl = pl.ds(pl.multiple_of(g * 16, 16), 16)
      slh = pl.ds(c * _DG + g * 16, 16)
      didx[s][sl] = ebuf[s][0, 1, slh]
      wsrc[s][sl] = wbuf[s][slh]
      return carry3

    lax.fori_loop(0, _DG // 16, per_g, 0)

  def per_l(l, carry):
    pltpu.sync_copy(zbuf, acc.at[pl.ds(pl.multiple_of(t * 640, 8), 640)])
    plsc.subcore_barrier()

    _idx_start(0, l, 0)
    _idx_start(1, l, 1)

    def pair(j, carry2):
      _idx_wait(0)

      @pl.when(j > 0)
      def _():
        _scatter_wait(0)

      _grab(0)
      _scatter_start(0)
      _idx_start(0, l, 2 * j + 2)
      _idx_wait(1)

      @pl.when(j > 0)
      def _():
        _scatter_wait(1)

      _grab(1)
      _scatter_start(1)

      @pl.when(j < _NPAIRS - 1)
      def _():
        _idx_start(1, l, 2 * j + 3)

      return carry2

    lax.fori_loop(0, _NPAIRS, pair, 0)
    _idx_wait(0)
    _scatter_wait(0)
    _grab(0)
    _scatter_start(0)
    _scatter_wait(0)
    _scatter_wait(1)
    plsc.subcore_barrier()
    ob = pl.multiple_of(c * (L * N) + l * N, 8)

    @pl.when(t < NS - 1)
    def _():
      o = pl.multiple_of(t * 640, 8)
      pltpu.sync_copy(acc.at[pl.ds(o, 640)], tout)
      pltpu.sync_copy(tout, out_hbm.at[pl.ds(ob + o, 640)])

    @pl.when(t == NS - 1)
    def _():
      pltpu.sync_copy(acc.at[pl.ds(9600, 400)], tout.at[pl.ds(0, 400)])
      pltpu.sync_copy(tout.at[pl.ds(0, 400)], out_hbm.at[pl.ds(ob + 9600, 400)])

    return carry

  lax.fori_loop(0, L, per_l, 0)


_deg_call = pl.kernel(
    _deg_body,
    out_type=jax.ShapeDtypeStruct((NC * L * N,), _f32),
    mesh=plsc.VectorSubcoreMesh(core_axis_name="c", subcore_axis_name="s",
                                num_cores=NC, num_subcores=NS),
    compiler_params=pltpu.CompilerParams(use_tc_tiling_on_sc=False),
    scratch_types=[
        pltpu.VMEM_SHARED((_NP,), _f32),
        pltpu.VMEM((1, 2, _CB), _i32),
        pltpu.VMEM((1, 2, _CB), _i32),
        pltpu.VMEM((_CB,), _f32),
        pltpu.VMEM((_CB,), _f32),
        pltpu.VMEM((_DG,), _i32),
        pltpu.VMEM((_DG,), _i32),
        pltpu.VMEM((_DG,), _f32),
        pltpu.VMEM((_DG,), _f32),
        pltpu.VMEM((640,), _f32),
        pltpu.VMEM((640,), _f32),
        pltpu.SemaphoreType.DMA,
        pltpu.SemaphoreType.DMA,
        pltpu.SemaphoreType.DMA,
        pltpu.SemaphoreType.DMA,
    ],
)

_EPT = E // NS
_CB = 800
_NCH = _EPT // _CB
_RPT = 640
_RLAST = N - (NS - 1) * _RPT


_bf16 = jnp.bfloat16
_NPAIRS = _NCH // 2


def _conv_body(tpa, tpb, epk, w_hbm, sa, sb, acc,
               ebufa, ebufb, wbufa, wbufb, didxa, didxb,
               gbufa, gbufb, sbufa, sbufb,
               isema, isemb, gsema, gsemb, ssema, ssemb):
  c = lax.axis_index("c")
  t = lax.axis_index("s")
  ebuf = (ebufa, ebufb)
  wbuf = (wbufa, wbufb)
  didx = (didxa, didxb)
  gbuf = (gbufa, gbufb)
  sbuf = (sbufa, sbufb)
  isem = (isema, isemb)
  gsem = (gsema, gsemb)
  ssem = (ssema, ssemb)

  def _row(l, k):
    return (l * NS + t) * _NCH + k

  def _idx_start(s, l, k):
    pltpu.async_copy(epk.at[pl.ds(_row(l, k), 1), :, :], ebuf[s], isem[s])
    eb = pl.multiple_of(l * E + t * _EPT + k * _CB, 8)
    pltpu.async_copy(w_hbm.at[pl.ds(eb, _CB)], wbuf[s], isem[s])

  def _idx_wait(s):
    pltpu.make_async_copy(epk.at[pl.ds(0, 1), :, :], ebuf[s], isem[s]).wait()
    pltpu.make_async_copy(w_hbm.at[pl.ds(0, _CB)], wbuf[s], isem[s]).wait()

  def _gather_start(s):
    @pl.when(c == 0)
    def _():
      pltpu.async_copy(tpa.at[ebuf[s].at[0, 0]], gbuf[s], gsem[s])

    @pl.when(c == 1)
    def _():
      pltpu.async_copy(tpb.at[ebuf[s].at[0, 0]], gbuf[s], gsem[s])

  def _gather_wait(s):
    @pl.when(c == 0)
    def _():
      pltpu.make_async_copy(tpa.at[ebuf[s].at[0, 0]], gbuf[s], gsem[s]).wait()

    @pl.when(c == 1)
    def _():
      pltpu.make_async_copy(tpb.at[ebuf[s].at[0, 0]], gbuf[s], gsem[s]).wait()

  def _scale(s):
    def per_g(g, carry3):
      sl = pl.ds(pl.multiple_of(g * 16, 16), 16)
      didx[s][sl] = ebuf[s][0, 1, sl]
      wv = wbuf[s][sl]
      for e16 in range(16):
        e = g * 16 + e16
        ws = jnp.broadcast_to(wv[e16], (HALF,)).astype(_bf16)
        sbuf[s][e, :] = gbuf[s][e, :] * ws
      return carry3

    lax.fori_loop(0, _CB // 16, per_g, 0)

  def _scatter_start(s):
    pltpu.async_copy(sbuf[s], acc.at[didx[s]], ssem[s], add=True)

  def _scatter_wait(s):
    pltpu.make_async_copy(sbuf[s], acc.at[didx[s]], ssem[s]).wait()

  def _stage(nrows):
    ra = pl.multiple_of(t * _RPT, 8)
    pltpu.sync_copy(sbufa.at[pl.ds(0, nrows), :], acc.at[pl.ds(ra, nrows), :])

  def _unstage(nrows, l):
    rb = pl.multiple_of(l * N + t * _RPT, 8)
    ra = pl.multiple_of(t * _RPT, 8)
    pltpu.sync_copy(acc.at[pl.ds(ra, nrows), :], sbufa.at[pl.ds(0, nrows), :])

    @pl.when(c == 0)
    def _():
      pltpu.sync_copy(sbufa.at[pl.ds(0, nrows), :], sa.at[pl.ds(rb, nrows), :])

    @pl.when(c == 1)
    def _():
      pltpu.sync_copy(sbufa.at[pl.ds(0, nrows), :], sb.at[pl.ds(rb, nrows), :])

  def per_l(l, carry):
    def zb(i, carry0):
      sbufa[i, :] = jnp.zeros((HALF,), _bf16)
      return carry0

    lax.fori_loop(0, _RPT, zb, 0)

    @pl.when(t < NS - 1)
    def _():
      _stage(_RPT)

    @pl.when(t == NS - 1)
    def _():
      _stage(_RLAST)

    plsc.subcore_barrier()

    _idx_start(0, l, 0)
    _idx_start(1, l, 1)
    _idx_wait(0)
    _gather_start(0)

    def pair(j, carry2):
      _gather_wait(0)

      @pl.when(j > 0)
      def _():
        _scatter_wait(0)

      _scale(0)
      _scatter_start(0)
      _idx_start(0, l, 2 * j + 2)
      _idx_wait(1)
      _gather_start(1)
      _gather_wait(1)

      @pl.when(j > 0)
      def _():
        _scatter_wait(1)

      _scale(1)
      _scatter_start(1)

      @pl.when(j < _NPAIRS - 1)
      def _():
        _idx_start(1, l, 2 * j + 3)

      _idx_wait(0)
      _gather_start(0)
      return carry2

    lax.fori_loop(0, _NPAIRS, pair, 0)
    _gather_wait(0)
    _scatter_wait(0)
    _scale(0)
    _scatter_start(0)
    _scatter_wait(0)
    _scatter_wait(1)
    plsc.subcore_barrier()

    @pl.when(t < NS - 1)
    def _():
      _unstage(_RPT, l)

    @pl.when(t == NS - 1)
    def _():
      _unstage(_RLAST, l)

    return carry

  lax.fori_loop(0, L, per_l, 0)


_conv_call = pl.kernel(
    _conv_body,
    out_type=[
        jax.ShapeDtypeStruct((L * N, HALF), _bf16),
        jax.ShapeDtypeStruct((L * N, HALF), _bf16),
    ],
    mesh=plsc.VectorSubcoreMesh(core_axis_name="c", subcore_axis_name="s",
                                num_cores=NC, num_subcores=NS),
    compiler_params=pltpu.CompilerParams(use_tc_tiling_on_sc=False),
    scratch_types=[
        pltpu.VMEM_SHARED((N, HALF), _bf16),
        pltpu.VMEM((1, 2, _CB), _i32),
        pltpu.VMEM((1, 2, _CB), _i32),
        pltpu.VMEM((_CB,), _f32),
        pltpu.VMEM((_CB,), _f32),
        pltpu.VMEM((_CB,), _i32),
        pltpu.VMEM((_CB,), _i32),
        pltpu.VMEM((_CB, HALF), _bf16),
        pltpu.VMEM((_CB, HALF), _bf16),
        pltpu.VMEM((_CB, HALF), _bf16),
        pltpu.VMEM((_CB, HALF), _bf16),
        pltpu.SemaphoreType.DMA,
        pltpu.SemaphoreType.DMA,
        pltpu.SemaphoreType.DMA,
        pltpu.SemaphoreType.DMA,
        pltpu.SemaphoreType.DMA,
        pltpu.SemaphoreType.DMA,
    ],
)

_BN = 2000
_NB = N // _BN


def _prep_block(degp_ref, x_ref, w1_ref, dis_ref, ta_ref, tb_ref,
                tab_ref, tbb_ref):
  deg = degp_ref[0, :, 0] + degp_ref[0, :, 1] + 1.0
  dis = lax.rsqrt(deg)
  dis_ref[0, :, 0] = dis
  tmat = jnp.dot(x_ref[0], w1_ref[...], preferred_element_type=_f32)
  ts = tmat * dis[:, None]
  ta_ref[0] = ts[:, :HALF]
  tb_ref[0] = ts[:, HALF:]
  tab_ref[0] = ts[:, :HALF].astype(jnp.bfloat16)
  tbb_ref[0] = ts[:, HALF:].astype(jnp.bfloat16)


def _prep_call(degp, x, w1):
  return pl.pallas_call(
      _prep_block,
      grid=(L, _NB),
      in_specs=[
          pl.BlockSpec((1, _BN, 2), lambda l, i: (l, i, 0)),
          pl.BlockSpec((1, _BN, D), lambda l, i: (l, i, 0)),
          pl.BlockSpec((D, D), lambda l, i: (0, 0)),
      ],
      out_specs=[
          pl.BlockSpec((1, _BN, 1), lambda l, i: (l, i, 0)),
          pl.BlockSpec((1, _BN, HALF), lambda l, i: (l, i, 0)),
          pl.BlockSpec((1, _BN, HALF), lambda l, i: (l, i, 0)),
          pl.BlockSpec((1, _BN, HALF), lambda l, i: (l, i, 0)),
          pl.BlockSpec((1, _BN, HALF), lambda l, i: (l, i, 0)),
      ],
      out_shape=[
          jax.ShapeDtypeStruct((L, N, 1), _f32),
          jax.ShapeDtypeStruct((L, N, HALF), _f32),
          jax.ShapeDtypeStruct((L, N, HALF), _f32),
          jax.ShapeDtypeStruct((L, N, HALF), jnp.bfloat16),
          jax.ShapeDtypeStruct((L, N, HALF), jnp.bfloat16),
      ],
  )(degp, x, w1)


def _mid_block(sa_ref, sb_ref, ta_ref, tb_ref, dis_ref, b_ref, w_ref,
               ta2_ref, tb2_ref, tab2_ref, tbb2_ref):
  d = dis_ref[0, :, 0]
  s_plus_t = jnp.concatenate(
      [sa_ref[0] + ta_ref[0], sb_ref[0] + tb_ref[0]], axis=1)
  h = jnp.maximum(d[:, None] * s_plus_t + b_ref[0][None, :], 0.0)
  t2 = jnp.dot(h, w_ref[...], preferred_element_type=_f32)
  ts = t2 * d[:, None]
  ta2_ref[0] = ts[:, :HALF]
  tb2_ref[0] = ts[:, HALF:]
  tab2_ref[0] = ts[:, :HALF].astype(jnp.bfloat16)
  tbb2_ref[0] = ts[:, HALF:].astype(jnp.bfloat16)


def _mid_call(sa, sb, ta, tb, dis, b, w):
  return pl.pallas_call(
      _mid_block,
      grid=(L, _NB),
      in_specs=[
          pl.BlockSpec((1, _BN, HALF), lambda l, i: (l, i, 0)),
          pl.BlockSpec((1, _BN, HALF), lambda l, i: (l, i, 0)),
          pl.BlockSpec((1, _BN, HALF), lambda l, i: (l, i, 0)),
          pl.BlockSpec((1, _BN, HALF), lambda l, i: (l, i, 0)),
          pl.BlockSpec((1, _BN, 1), lambda l, i: (l, i, 0)),
          pl.BlockSpec((1, D), lambda l, i: (0, 0)),
          pl.BlockSpec((D, D), lambda l, i: (0, 0)),
      ],
      out_specs=[
          pl.BlockSpec((1, _BN, HALF), lambda l, i: (l, i, 0)),
          pl.BlockSpec((1, _BN, HALF), lambda l, i: (l, i, 0)),
          pl.BlockSpec((1, _BN, HALF), lambda l, i: (l, i, 0)),
          pl.BlockSpec((1, _BN, HALF), lambda l, i: (l, i, 0)),
      ],
      out_shape=[
          jax.ShapeDtypeStruct((L, N, HALF), _f32),
          jax.ShapeDtypeStruct((L, N, HALF), _f32),
          jax.ShapeDtypeStruct((L, N, HALF), jnp.bfloat16),
          jax.ShapeDtypeStruct((L, N, HALF), jnp.bfloat16),
      ],
  )(sa, sb, ta, tb, dis, b, w)


def _fin_block(sa_ref, sb_ref, ta_ref, tb_ref, dis_ref, b_ref, pp_ref):
  d = dis_ref[0, :, 0]
  s_plus_t = jnp.concatenate(
      [sa_ref[0] + ta_ref[0], sb_ref[0] + tb_ref[0]], axis=1)
  z = jnp.maximum(d[:, None] * s_plus_t + b_ref[0][None, :], 0.0)
  pp_ref[0, 0, 0, :] = jnp.sum(z, axis=0)


def _fin_call(sa, sb, ta, tb, dis, b):
  return pl.pallas_call(
      _fin_block,
      grid=(L, _NB),
      in_specs=[
          pl.BlockSpec((1, _BN, HALF), lambda l, i: (l, i, 0)),
          pl.BlockSpec((1, _BN, HALF), lambda l, i: (l, i, 0)),
          pl.BlockSpec((1, _BN, HALF), lambda l, i: (l, i, 0)),
          pl.BlockSpec((1, _BN, HALF), lambda l, i: (l, i, 0)),
          pl.BlockSpec((1, _BN, 1), lambda l, i: (l, i, 0)),
          pl.BlockSpec((1, D), lambda l, i: (0, 0)),
      ],
      out_specs=[pl.BlockSpec((1, 1, 1, D), lambda l, i: (l, i, 0, 0))],
      out_shape=[jax.ShapeDtypeStruct((L, _NB, 1, D), _f32)],
  )(sa, sb, ta, tb, dis, b)[0]


def _rnn_block(pp_ref, wih_t_ref, bih_ref, whh_t_ref, bhh_ref, wout_ref,
               bout_ref, out_ref):
  seq = jnp.sum(pp_ref[...], axis=(1, 2)) * (1.0 / N)
  h = jnp.zeros((1, D), _f32)
  hs = []
  for i in range(L):
    xt = lax.slice(seq, (i, 0), (i + 1, D))
    h = jnp.tanh(
        jnp.dot(xt, wih_t_ref[...], preferred_element_type=_f32)
        + bih_ref[...]
        + jnp.dot(h, whh_t_ref[...], preferred_element_type=_f32)
        + bhh_ref[...])
    hs.append(h)
  hsm = jnp.concatenate(hs, axis=0)
  logits = jnp.dot(hsm, wout_ref[...], preferred_element_type=_f32)
  out_ref[...] = jax.nn.sigmoid(logits + bout_ref[...])


def _rnn_call(pp, wih_t, bih, whh_t, bhh, wout, bout):
  return pl.pallas_call(
      _rnn_block,
      out_shape=jax.ShapeDtypeStruct((L, 2), _f32),
  )(pp, wih_t, bih, whh_t, bhh, wout, bout)


def kernel(x, edge_index, edge_weight, W1, b1, W2, b2, W3, b3,
           W_ih, b_ih, W_hh, b_hh, W_out, b_out):
  loff = (jnp.arange(L, dtype=_i32) * N)[:, None]
  w = edge_weight.reshape(L * E)
  srcr = (edge_index[:, 0, :] + loff).reshape(L, NS, _NCH, _CB)
  dstr = edge_index[:, 1, :].reshape(L, NS, _NCH, _CB)
  epk = jnp.stack([srcr, dstr], axis=3).reshape(L * NS * _NCH, 2, _CB)

  degp = _deg_call(epk, w)
  degt = jnp.transpose(degp.reshape(NC, L, N), (1, 2, 0))

  dis, ta, tb, tab, tbb = _prep_call(degt, x, W1)

  def flat(a):
    return a.reshape(L * N, HALF)

  def unflat(a):
    return a.reshape(L, N, HALF)

  sa, sb = _conv_call(flat(tab), flat(tbb), epk, w)
  ta, tb, tab, tbb = _mid_call(unflat(sa), unflat(sb), ta, tb, dis,
                               b1.reshape(1, D), W2)
  sa, sb = _conv_call(flat(tab), flat(tbb), epk, w)
  ta, tb, tab, tbb = _mid_call(unflat(sa), unflat(sb), ta, tb, dis,
                               b2.reshape(1, D), W3)
  sa, sb = _conv_call(flat(tab), flat(tbb), epk, w)
  pp = _fin_call(unflat(sa), unflat(sb), ta, tb, dis, b3.reshape(1, D))

  return _rnn_call(pp, W_ih.T, b_ih.reshape(1, D), W_hh.T,
                   b_hh.reshape(1, D), W_out, b_out.reshape(1, 2))


# speedup vs baseline: 31.2752x; 1.1624x over previous
import functools

import jax
import jax.numpy as jnp
from jax import lax
from jax.experimental import pallas as pl
from jax.experimental.pallas import tpu as pltpu
from jax.experimental.pallas import tpu_sc as plsc

L, N, D = 8, 10000, 128
E = 320000
HALF = D // 2
NC, NS = 2, 16

_f32 = jnp.float32
_i32 = jnp.int32

_DEG_EPT = E // (NC * NS)
_DEG_CB = 1000
_DEG_NCH = _DEG_EPT // _DEG_CB

_NP = 10240


_CB = 800
_EPT = E // NS
_NCH = _EPT // _CB
_DG = _CB // NC


def _deg_body(epk, w_hbm, out_hbm, acc,
              ebufa, ebufb, wbufa, wbufb, didxa, didxb, wsrca, wsrcb,
              zbuf, tout, isema, isemb, ssema, ssemb):
  c = lax.axis_index("c")
  t = lax.axis_index("s")
  ebuf = (ebufa, ebufb)
  wbuf = (wbufa, wbufb)
  didx = (didxa, didxb)
  wsrc = (wsrca, wsrcb)
  isem = (isema, isemb)
  ssem = (ssema, ssemb)
  for i in range(640 // 16):
    zbuf[pl.ds(i * 16, 16)] = jnp.zeros((16,), _f32)

  def _idx_start(s, l, k):
    row = (l * NS + t) * _NCH + k
    pltpu.async_copy(epk.at[pl.ds(row, 1), :, :], ebuf[s], isem[s])
    eb = pl.multiple_of(l * E + t * _EPT + k * _CB, 8)
    pltpu.async_copy(w_hbm.at[pl.ds(eb, _CB)], wbuf[s], isem[s])

  def _idx_wait(s):
    pltpu.make_async_copy(epk.at[pl.ds(0, 1), :, :], ebuf[s], isem[s]).wait()
    pltpu.make_async_copy(w_hbm.at[pl.ds(0, _CB)], wbuf[s], isem[s]).wait()

  def _scatter_start(s):
    pltpu.async_copy(wsrc[s], acc.at[didx[s]], ssem[s], add=True)

  def _scatter_wait(s):
    pltpu.make_async_copy(wsrc[s], acc.at[didx[s]], ssem[s]).wait()

  def _grab(s):
    def per_g(g, carry3):
      sl = pl.ds(pl.multiple_of(g * 16, 16), 16)
      slh = pl.ds(c * _DG + g * 16, 16)
      didx[s][sl] = ebuf[s][0, 1, slh]
      wsrc[s][sl] = wbuf[s][slh]
      return carry3

    lax.fori_loop(0, _DG // 16, per_g, 0)

  def per_l(l, carry):
    pltpu.sync_copy(zbuf, acc.at[pl.ds(pl.multiple_of(t * 640, 8), 640)])
    plsc.subcore_barrier()

    _idx_start(0, l, 0)
    _idx_start(1, l, 1)

    def pair(j, carry2):
      _idx_wait(0)

      @pl.when(j > 0)
      def _():
        _scatter_wait(0)

      _grab(0)
      _scatter_start(0)
      _idx_start(0, l, 2 * j + 2)
      _idx_wait(1)

      @pl.when(j > 0)
      def _():
        _scatter_wait(1)

      _grab(1)
      _scatter_start(1)

      @pl.when(j < _NPAIRS - 1)
      def _():
        _idx_start(1, l, 2 * j + 3)

      return carry2

    lax.fori_loop(0, _NPAIRS, pair, 0)
    _idx_wait(0)
    _scatter_wait(0)
    _grab(0)
    _scatter_start(0)
    _scatter_wait(0)
    _scatter_wait(1)
    plsc.subcore_barrier()
    ob = pl.multiple_of(c * (L * N) + l * N, 8)

    @pl.when(t < NS - 1)
    def _():
      o = pl.multiple_of(t * 640, 8)
      pltpu.sync_copy(acc.at[pl.ds(o, 640)], tout)
      pltpu.sync_copy(tout, out_hbm.at[pl.ds(ob + o, 640)])

    @pl.when(t == NS - 1)
    def _():
      pltpu.sync_copy(acc.at[pl.ds(9600, 400)], tout.at[pl.ds(0, 400)])
      pltpu.sync_copy(tout.at[pl.ds(0, 400)], out_hbm.at[pl.ds(ob + 9600, 400)])

    return carry

  lax.fori_loop(0, L, per_l, 0)


_deg_call = pl.kernel(
    _deg_body,
    out_type=jax.ShapeDtypeStruct((NC * L * N,), _f32),
    mesh=plsc.VectorSubcoreMesh(core_axis_name="c", subcore_axis_name="s",
                                num_cores=NC, num_subcores=NS),
    compiler_params=pltpu.CompilerParams(use_tc_tiling_on_sc=False),
    scratch_types=[
        pltpu.VMEM_SHARED((_NP,), _f32),
        pltpu.VMEM((1, 2, _CB), _i32),
        pltpu.VMEM((1, 2, _CB), _i32),
        pltpu.VMEM((_CB,), _f32),
        pltpu.VMEM((_CB,), _f32),
        pltpu.VMEM((_DG,), _i32),
        pltpu.VMEM((_DG,), _i32),
        pltpu.VMEM((_DG,), _f32),
        pltpu.VMEM((_DG,), _f32),
        pltpu.VMEM((640,), _f32),
        pltpu.VMEM((640,), _f32),
        pltpu.SemaphoreType.DMA,
        pltpu.SemaphoreType.DMA,
        pltpu.SemaphoreType.DMA,
        pltpu.SemaphoreType.DMA,
    ],
)

_EPT = E // NS
_CB = 800
_NCH = _EPT // _CB
_RPT = 640
_RLAST = N - (NS - 1) * _RPT


_bf16 = jnp.bfloat16
_NPAIRS = _NCH // 2


def _conv_body(tpa, tpb, epk, w_hbm, sa, sb, acc,
               ebufa, ebufb, wbufa, wbufb, didxa, didxb,
               gbufa, gbufb, sbufa, sbufb,
               isema, isemb, gsema, gsemb, ssema, ssemb):
  c = lax.axis_index("c")
  t = lax.axis_index("s")
  ebuf = (ebufa, ebufb)
  wbuf = (wbufa, wbufb)
  didx = (didxa, didxb)
  gbuf = (gbufa, gbufb)
  sbuf = (sbufa, sbufb)
  isem = (isema, isemb)
  gsem = (gsema, gsemb)
  ssem = (ssema, ssemb)

  def _row(l, k):
    return (l * NS + t) * _NCH + k

  def _idx_start(s, l, k):
    pltpu.async_copy(epk.at[pl.ds(_row(l, k), 1), :, :], ebuf[s], isem[s])
    eb = pl.multiple_of(l * E + t * _EPT + k * _CB, 8)
    pltpu.async_copy(w_hbm.at[pl.ds(eb, _CB)], wbuf[s], isem[s])

  def _idx_wait(s):
    pltpu.make_async_copy(epk.at[pl.ds(0, 1), :, :], ebuf[s], isem[s]).wait()
    pltpu.make_async_copy(w_hbm.at[pl.ds(0, _CB)], wbuf[s], isem[s]).wait()

  def _gather_start(s):
    @pl.when(c == 0)
    def _():
      pltpu.async_copy(tpa.at[ebuf[s].at[0, 0]], gbuf[s], gsem[s])

    @pl.when(c == 1)
    def _():
      pltpu.async_copy(tpb.at[ebuf[s].at[0, 0]], gbuf[s], gsem[s])

  def _gather_wait(s):
    @pl.when(c == 0)
    def _():
      pltpu.make_async_copy(tpa.at[ebuf[s].at[0, 0]], gbuf[s], gsem[s]).wait()

    @pl.when(c == 1)
    def _():
      pltpu.make_async_copy(tpb.at[ebuf[s].at[0, 0]], gbuf[s], gsem[s]).wait()

  def _scale(s):
    def per_g(g, carry3):
      sl = pl.ds(pl.multiple_of(g * 16, 16), 16)
      didx[s][sl] = ebuf[s][0, 1, sl]
      wv = wbuf[s][sl]
      for e16 in range(16):
        e = g * 16 + e16
        ws = jnp.broadcast_to(wv[e16], (HALF,)).astype(_bf16)
        sbuf[s][e, :] = gbuf[s][e, :] * ws
      return carry3

    lax.fori_loop(0, _CB // 16, per_g, 0)

  def _scatter_start(s):
    pltpu.async_copy(sbuf[s], acc.at[didx[s]], ssem[s], add=True)

  def _scatter_wait(s):
    pltpu.make_async_copy(sbuf[s], acc.at[didx[s]], ssem[s]).wait()

  def _stage(nrows):
    ra = pl.multiple_of(t * _RPT, 8)
    pltpu.sync_copy(sbufa.at[pl.ds(0, nrows), :], acc.at[pl.ds(ra, nrows), :])

  def _unstage(nrows, l):
    rb = pl.multiple_of(l * N + t * _RPT, 8)
    ra = pl.multiple_of(t * _RPT, 8)
    pltpu.sync_copy(acc.at[pl.ds(ra, nrows), :], sbufa.at[pl.ds(0, nrows), :])

    @pl.when(c == 0)
    def _():
      pltpu.sync_copy(sbufa.at[pl.ds(0, nrows), :], sa.at[pl.ds(rb, nrows), :])

    @pl.when(c == 1)
    def _():
      pltpu.sync_copy(sbufa.at[pl.ds(0, nrows), :], sb.at[pl.ds(rb, nrows), :])

  def per_l(l, carry):
    def zb(i, carry0):
      sbufa[i, :] = jnp.zeros((HALF,), _bf16)
      return carry0

    lax.fori_loop(0, _RPT, zb, 0)

    @pl.when(t < NS - 1)
    def _():
      _stage(_RPT)

    @pl.when(t == NS - 1)
    def _():
      _stage(_RLAST)

    plsc.subcore_barrier()

    _idx_start(0, l, 0)
    _idx_start(1, l, 1)
    _idx_wait(0)
    _gather_start(0)

    def pair(j, carry2):
      _idx_wait(1)
      _gather_start(1)
      _gather_wait(0)

      @pl.when(j > 0)
      def _():
        _scatter_wait(0)

      _scale(0)
      _scatter_start(0)
      _idx_start(0, l, 2 * j + 2)
      _idx_wait(0)
      _gather_start(0)
      _gather_wait(1)

      @pl.when(j > 0)
      def _():
        _scatter_wait(1)

      _scale(1)
      _scatter_start(1)

      @pl.when(j < _NPAIRS - 1)
      def _():
        _idx_start(1, l, 2 * j + 3)

      return carry2

    lax.fori_loop(0, _NPAIRS, pair, 0)
    _gather_wait(0)
    _scatter_wait(0)
    _scale(0)
    _scatter_start(0)
    _scatter_wait(0)
    _scatter_wait(1)
    plsc.subcore_barrier()

    @pl.when(t < NS - 1)
    def _():
      _unstage(_RPT, l)

    @pl.when(t == NS - 1)
    def _():
      _unstage(_RLAST, l)

    return carry

  lax.fori_loop(0, L, per_l, 0)


_conv_call = pl.kernel(
    _conv_body,
    out_type=[
        jax.ShapeDtypeStruct((L * N, HALF), _bf16),
        jax.ShapeDtypeStruct((L * N, HALF), _bf16),
    ],
    mesh=plsc.VectorSubcoreMesh(core_axis_name="c", subcore_axis_name="s",
                                num_cores=NC, num_subcores=NS),
    compiler_params=pltpu.CompilerParams(use_tc_tiling_on_sc=False),
    scratch_types=[
        pltpu.VMEM_SHARED((N, HALF), _bf16),
        pltpu.VMEM((1, 2, _CB), _i32),
        pltpu.VMEM((1, 2, _CB), _i32),
        pltpu.VMEM((_CB,), _f32),
        pltpu.VMEM((_CB,), _f32),
        pltpu.VMEM((_CB,), _i32),
        pltpu.VMEM((_CB,), _i32),
        pltpu.VMEM((_CB, HALF), _bf16),
        pltpu.VMEM((_CB, HALF), _bf16),
        pltpu.VMEM((_CB, HALF), _bf16),
        pltpu.VMEM((_CB, HALF), _bf16),
        pltpu.SemaphoreType.DMA,
        pltpu.SemaphoreType.DMA,
        pltpu.SemaphoreType.DMA,
        pltpu.SemaphoreType.DMA,
        pltpu.SemaphoreType.DMA,
        pltpu.SemaphoreType.DMA,
    ],
)

_BN = 2000
_NB = N // _BN


def _prep_block(degp_ref, x_ref, w1_ref, dis_ref, ta_ref, tb_ref,
                tab_ref, tbb_ref):
  deg = degp_ref[0, :, 0] + degp_ref[0, :, 1] + 1.0
  dis = lax.rsqrt(deg)
  dis_ref[0, :, 0] = dis
  tmat = jnp.dot(x_ref[0], w1_ref[...], preferred_element_type=_f32)
  ts = tmat * dis[:, None]
  ta_ref[0] = ts[:, :HALF]
  tb_ref[0] = ts[:, HALF:]
  tab_ref[0] = ts[:, :HALF].astype(jnp.bfloat16)
  tbb_ref[0] = ts[:, HALF:].astype(jnp.bfloat16)


def _prep_call(degp, x, w1):
  return pl.pallas_call(
      _prep_block,
      grid=(L, _NB),
      in_specs=[
          pl.BlockSpec((1, _BN, 2), lambda l, i: (l, i, 0)),
          pl.BlockSpec((1, _BN, D), lambda l, i: (l, i, 0)),
          pl.BlockSpec((D, D), lambda l, i: (0, 0)),
      ],
      out_specs=[
          pl.BlockSpec((1, _BN, 1), lambda l, i: (l, i, 0)),
          pl.BlockSpec((1, _BN, HALF), lambda l, i: (l, i, 0)),
          pl.BlockSpec((1, _BN, HALF), lambda l, i: (l, i, 0)),
          pl.BlockSpec((1, _BN, HALF), lambda l, i: (l, i, 0)),
          pl.BlockSpec((1, _BN, HALF), lambda l, i: (l, i, 0)),
      ],
      out_shape=[
          jax.ShapeDtypeStruct((L, N, 1), _f32),
          jax.ShapeDtypeStruct((L, N, HALF), _f32),
          jax.ShapeDtypeStruct((L, N, HALF), _f32),
          jax.ShapeDtypeStruct((L, N, HALF), jnp.bfloat16),
          jax.ShapeDtypeStruct((L, N, HALF), jnp.bfloat16),
      ],
  )(degp, x, w1)


def _mid_block(sa_ref, sb_ref, ta_ref, tb_ref, dis_ref, b_ref, w_ref,
               ta2_ref, tb2_ref, tab2_ref, tbb2_ref):
  d = dis_ref[0, :, 0]
  s_plus_t = jnp.concatenate(
      [sa_ref[0] + ta_ref[0], sb_ref[0] + tb_ref[0]], axis=1)
  h = jnp.maximum(d[:, None] * s_plus_t + b_ref[0][None, :], 0.0)
  t2 = jnp.dot(h, w_ref[...], preferred_element_type=_f32)
  ts = t2 * d[:, None]
  ta2_ref[0] = ts[:, :HALF]
  tb2_ref[0] = ts[:, HALF:]
  tab2_ref[0] = ts[:, :HALF].astype(jnp.bfloat16)
  tbb2_ref[0] = ts[:, HALF:].astype(jnp.bfloat16)


def _mid_call(sa, sb, ta, tb, dis, b, w):
  return pl.pallas_call(
      _mid_block,
      grid=(L, _NB),
      in_specs=[
          pl.BlockSpec((1, _BN, HALF), lambda l, i: (l, i, 0)),
          pl.BlockSpec((1, _BN, HALF), lambda l, i: (l, i, 0)),
          pl.BlockSpec((1, _BN, HALF), lambda l, i: (l, i, 0)),
          pl.BlockSpec((1, _BN, HALF), lambda l, i: (l, i, 0)),
          pl.BlockSpec((1, _BN, 1), lambda l, i: (l, i, 0)),
          pl.BlockSpec((1, D), lambda l, i: (0, 0)),
          pl.BlockSpec((D, D), lambda l, i: (0, 0)),
      ],
      out_specs=[
          pl.BlockSpec((1, _BN, HALF), lambda l, i: (l, i, 0)),
          pl.BlockSpec((1, _BN, HALF), lambda l, i: (l, i, 0)),
          pl.BlockSpec((1, _BN, HALF), lambda l, i: (l, i, 0)),
          pl.BlockSpec((1, _BN, HALF), lambda l, i: (l, i, 0)),
      ],
      out_shape=[
          jax.ShapeDtypeStruct((L, N, HALF), _f32),
          jax.ShapeDtypeStruct((L, N, HALF), _f32),
          jax.ShapeDtypeStruct((L, N, HALF), jnp.bfloat16),
          jax.ShapeDtypeStruct((L, N, HALF), jnp.bfloat16),
      ],
  )(sa, sb, ta, tb, dis, b, w)


def _fin_block(sa_ref, sb_ref, ta_ref, tb_ref, dis_ref, b_ref, pp_ref):
  d = dis_ref[0, :, 0]
  s_plus_t = jnp.concatenate(
      [sa_ref[0] + ta_ref[0], sb_ref[0] + tb_ref[0]], axis=1)
  z = jnp.maximum(d[:, None] * s_plus_t + b_ref[0][None, :], 0.0)
  pp_ref[0, 0, 0, :] = jnp.sum(z, axis=0)


def _fin_call(sa, sb, ta, tb, dis, b):
  return pl.pallas_call(
      _fin_block,
      grid=(L, _NB),
      in_specs=[
          pl.BlockSpec((1, _BN, HALF), lambda l, i: (l, i, 0)),
          pl.BlockSpec((1, _BN, HALF), lambda l, i: (l, i, 0)),
          pl.BlockSpec((1, _BN, HALF), lambda l, i: (l, i, 0)),
          pl.BlockSpec((1, _BN, HALF), lambda l, i: (l, i, 0)),
          pl.BlockSpec((1, _BN, 1), lambda l, i: (l, i, 0)),
          pl.BlockSpec((1, D), lambda l, i: (0, 0)),
      ],
      out_specs=[pl.BlockSpec((1, 1, 1, D), lambda l, i: (l, i, 0, 0))],
      out_shape=[jax.ShapeDtypeStruct((L, _NB, 1, D), _f32)],
  )(sa, sb, ta, tb, dis, b)[0]


def _rnn_block(pp_ref, wih_t_ref, bih_ref, whh_t_ref, bhh_ref, wout_ref,
               bout_ref, out_ref):
  seq = jnp.sum(pp_ref[...], axis=(1, 2)) * (1.0 / N)
  h = jnp.zeros((1, D), _f32)
  hs = []
  for i in range(L):
    xt = lax.slice(seq, (i, 0), (i + 1, D))
    h = jnp.tanh(
        jnp.dot(xt, wih_t_ref[...], preferred_element_type=_f32)
        + bih_ref[...]
        + jnp.dot(h, whh_t_ref[...], preferred_element_type=_f32)
        + bhh_ref[...])
    hs.append(h)
  hsm = jnp.concatenate(hs, axis=0)
  logits = jnp.dot(hsm, wout_ref[...], preferred_element_type=_f32)
  out_ref[...] = jax.nn.sigmoid(logits + bout_ref[...])


def _rnn_call(pp, wih_t, bih, whh_t, bhh, wout, bout):
  return pl.pallas_call(
      _rnn_block,
      out_shape=jax.ShapeDtypeStruct((L, 2), _f32),
  )(pp, wih_t, bih, whh_t, bhh, wout, bout)


def kernel(x, edge_index, edge_weight, W1, b1, W2, b2, W3, b3,
           W_ih, b_ih, W_hh, b_hh, W_out, b_out):
  loff = (jnp.arange(L, dtype=_i32) * N)[:, None]
  w = edge_weight.reshape(L * E)
  srcr = (edge_index[:, 0, :] + loff).reshape(L, NS, _NCH, _CB)
  dstr = edge_index[:, 1, :].reshape(L, NS, _NCH, _CB)
  epk = jnp.stack([srcr, dstr], axis=3).reshape(L * NS * _NCH, 2, _CB)

  degp = _deg_call(epk, w)
  degt = jnp.transpose(degp.reshape(NC, L, N), (1, 2, 0))

  dis, ta, tb, tab, tbb = _prep_call(degt, x, W1)

  def flat(a):
    return a.reshape(L * N, HALF)

  def unflat(a):
    return a.reshape(L, N, HALF)

  sa, sb = _conv_call(flat(tab), flat(tbb), epk, w)
  ta, tb, tab, tbb = _mid_call(unflat(sa), unflat(sb), ta, tb, dis,
                               b1.reshape(1, D), W2)
  sa, sb = _conv_call(flat(tab), flat(tbb), epk, w)
  ta, tb, tab, tbb = _mid_call(unflat(sa), unflat(sb), ta, tb, dis,
                               b2.reshape(1, D), W3)
  sa, sb = _conv_call(flat(tab), flat(tbb), epk, w)
  pp = _fin_call(unflat(sa), unflat(sb), ta, tb, dis, b3.reshape(1, D))

  return _rnn_call(pp, W_ih.T, b_ih.reshape(1, D), W_hh.T,
                   b_hh.reshape(1, D), W_out, b_out.reshape(1, 2))


# speedup vs baseline: 48.3582x; 1.5462x over previous
import functools

import jax
import jax.numpy as jnp
from jax import lax
from jax.experimental import pallas as pl
from jax.experimental.pallas import tpu as pltpu
from jax.experimental.pallas import tpu_sc as plsc

L, N, D = 8, 10000, 128
E = 320000
HALF = D // 2
NC, NS = 2, 16

_f32 = jnp.float32
_i32 = jnp.int32

_DEG_EPT = E // (NC * NS)
_DEG_CB = 1000
_DEG_NCH = _DEG_EPT // _DEG_CB

_NP = 10240


_CB = 800
_EPT = E // NS
_NCH = _EPT // _CB
_DG = _CB // NC


def _deg_body(epk, w_hbm, out_hbm, acc,
              ebufa, ebufb, wbufa, wbufb, didxa, didxb, wsrca, wsrcb,
              zbuf, tout, isema, isemb, ssema, ssemb):
  c = lax.axis_index("c")
  t = lax.axis_index("s")
  ebuf = (ebufa, ebufb)
  wbuf = (wbufa, wbufb)
  didx = (didxa, didxb)
  wsrc = (wsrca, wsrcb)
  isem = (isema, isemb)
  ssem = (ssema, ssemb)
  for i in range(640 // 16):
    zbuf[pl.ds(i * 16, 16)] = jnp.zeros((16,), _f32)

  def _idx_start(s, l, k):
    row = (l * NS + t) * _NCH + k
    pltpu.async_copy(epk.at[pl.ds(row, 1), :, :], ebuf[s], isem[s])
    eb = pl.multiple_of(l * E + t * _EPT + k * _CB, 8)
    pltpu.async_copy(w_hbm.at[pl.ds(eb, _CB)], wbuf[s], isem[s])

  def _idx_wait(s):
    pltpu.make_async_copy(epk.at[pl.ds(0, 1), :, :], ebuf[s], isem[s]).wait()
    pltpu.make_async_copy(w_hbm.at[pl.ds(0, _CB)], wbuf[s], isem[s]).wait()

  def _scatter_start(s):
    pltpu.async_copy(wsrc[s], acc.at[didx[s]], ssem[s], add=True)

  def _scatter_wait(s):
    pltpu.make_async_copy(wsrc[s], acc.at[didx[s]], ssem[s]).wait()

  def _grab(s):
    def per_g(g, carry3):
      sl = pl.ds(pl.multiple_of(g * 16, 16), 16)
      slh = pl.ds(c * _DG + g * 16, 16)
      didx[s][sl] = ebuf[s][0, 1, slh]
      wsrc[s][sl] = wbuf[s][slh]
      return carry3

    lax.fori_loop(0, _DG // 16, per_g, 0)

  def per_l(l, carry):
    pltpu.sync_copy(zbuf, acc.at[pl.ds(pl.multiple_of(t * 640, 8), 640)])
    plsc.subcore_barrier()

    _idx_start(0, l, 0)
    _idx_start(1, l, 1)

    def pair(j, carry2):
      _idx_wait(0)

      @pl.when(j > 0)
      def _():
        _scatter_wait(0)

      _grab(0)
      _scatter_start(0)
      _idx_start(0, l, 2 * j + 2)
      _idx_wait(1)

      @pl.when(j > 0)
      def _():
        _scatter_wait(1)

      _grab(1)
      _scatter_start(1)

      @pl.when(j < _NPAIRS - 1)
      def _():
        _idx_start(1, l, 2 * j + 3)

      return carry2

    lax.fori_loop(0, _NPAIRS, pair, 0)
    _idx_wait(0)
    _scatter_wait(0)
    _grab(0)
    _scatter_start(0)
    _scatter_wait(0)
    _scatter_wait(1)
    plsc.subcore_barrier()
    ob = pl.multiple_of(c * (L * N) + l * N, 8)

    @pl.when(t < NS - 1)
    def _():
      o = pl.multiple_of(t * 640, 8)
      pltpu.sync_copy(acc.at[pl.ds(o, 640)], tout)
      pltpu.sync_copy(tout, out_hbm.at[pl.ds(ob + o, 640)])

    @pl.when(t == NS - 1)
    def _():
      pltpu.sync_copy(acc.at[pl.ds(9600, 400)], tout.at[pl.ds(0, 400)])
      pltpu.sync_copy(tout.at[pl.ds(0, 400)], out_hbm.at[pl.ds(ob + 9600, 400)])

    return carry

  lax.fori_loop(0, L, per_l, 0)


_deg_call = pl.kernel(
    _deg_body,
    out_type=jax.ShapeDtypeStruct((NC * L * N,), _f32),
    mesh=plsc.VectorSubcoreMesh(core_axis_name="c", subcore_axis_name="s",
                                num_cores=NC, num_subcores=NS),
    compiler_params=pltpu.CompilerParams(use_tc_tiling_on_sc=False),
    scratch_types=[
        pltpu.VMEM_SHARED((_NP,), _f32),
        pltpu.VMEM((1, 2, _CB), _i32),
        pltpu.VMEM((1, 2, _CB), _i32),
        pltpu.VMEM((_CB,), _f32),
        pltpu.VMEM((_CB,), _f32),
        pltpu.VMEM((_DG,), _i32),
        pltpu.VMEM((_DG,), _i32),
        pltpu.VMEM((_DG,), _f32),
        pltpu.VMEM((_DG,), _f32),
        pltpu.VMEM((640,), _f32),
        pltpu.VMEM((640,), _f32),
        pltpu.SemaphoreType.DMA,
        pltpu.SemaphoreType.DMA,
        pltpu.SemaphoreType.DMA,
        pltpu.SemaphoreType.DMA,
    ],
)

_EPT = E // NS
_CB = 800
_NCH = _EPT // _CB
_RPT = 640
_RLAST = N - (NS - 1) * _RPT


_bf16 = jnp.bfloat16
_NPAIRS = _NCH // 2


def _conv_body(tpa, tpb, epk, w_hbm, sa, sb, acc,
               ebufa, ebufb, wbufa, wbufb, didxa, didxb,
               gbufa, gbufb, sbufa, sbufb,
               isema, isemb, gsema, gsemb, ssema, ssemb):
  c = lax.axis_index("c")
  t = lax.axis_index("s")
  ebuf = (ebufa, ebufb)
  wbuf = (wbufa, wbufb)
  didx = (didxa, didxb)
  gbuf = (gbufa, gbufb)
  sbuf = (sbufa, sbufb)
  isem = (isema, isemb)
  gsem = (gsema, gsemb)
  ssem = (ssema, ssemb)

  def _row(l, k):
    return (l * NS + t) * _NCH + k

  def _idx_start(s, l, k):
    pltpu.async_copy(epk.at[pl.ds(_row(l, k), 1), :, :], ebuf[s], isem[s])
    eb = pl.multiple_of(l * E + t * _EPT + k * _CB, 8)
    pltpu.async_copy(w_hbm.at[pl.ds(eb, _CB)], wbuf[s], isem[s])

  def _idx_wait(s):
    pltpu.make_async_copy(epk.at[pl.ds(0, 1), :, :], ebuf[s], isem[s]).wait()
    pltpu.make_async_copy(w_hbm.at[pl.ds(0, _CB)], wbuf[s], isem[s]).wait()

  def _gather_start(s):
    @pl.when(c == 0)
    def _():
      pltpu.async_copy(tpa.at[ebuf[s].at[0, 0]], gbuf[s], gsem[s])

    @pl.when(c == 1)
    def _():
      pltpu.async_copy(tpb.at[ebuf[s].at[0, 0]], gbuf[s], gsem[s])

  def _gather_wait(s):
    @pl.when(c == 0)
    def _():
      pltpu.make_async_copy(tpa.at[ebuf[s].at[0, 0]], gbuf[s], gsem[s]).wait()

    @pl.when(c == 1)
    def _():
      pltpu.make_async_copy(tpb.at[ebuf[s].at[0, 0]], gbuf[s], gsem[s]).wait()

  def _scale(s):
    @plsc.parallel_loop(0, _CB // 16, unroll=4)
    def per_g(g):
      sl = pl.ds(pl.multiple_of(g * 16, 16), 16)
      didx[s][sl] = ebuf[s][0, 1, sl]
      wv = wbuf[s][sl]
      for e16 in range(16):
        e = g * 16 + e16
        ws = jnp.broadcast_to(wv[e16], (HALF,)).astype(_bf16)
        sbuf[s][e, :] = gbuf[s][e, :] * ws

  def _scatter_start(s):
    pltpu.async_copy(sbuf[s], acc.at[didx[s]], ssem[s], add=True)

  def _scatter_wait(s):
    pltpu.make_async_copy(sbuf[s], acc.at[didx[s]], ssem[s]).wait()

  def _stage(nrows):
    ra = pl.multiple_of(t * _RPT, 8)
    pltpu.sync_copy(sbufa.at[pl.ds(0, nrows), :], acc.at[pl.ds(ra, nrows), :])

  def _unstage(nrows, l):
    rb = pl.multiple_of(l * N + t * _RPT, 8)
    ra = pl.multiple_of(t * _RPT, 8)
    pltpu.sync_copy(acc.at[pl.ds(ra, nrows), :], sbufa.at[pl.ds(0, nrows), :])

    @pl.when(c == 0)
    def _():
      pltpu.sync_copy(sbufa.at[pl.ds(0, nrows), :], sa.at[pl.ds(rb, nrows), :])

    @pl.when(c == 1)
    def _():
      pltpu.sync_copy(sbufa.at[pl.ds(0, nrows), :], sb.at[pl.ds(rb, nrows), :])

  def per_l(l, carry):
    def zb(i, carry0):
      sbufa[i, :] = jnp.zeros((HALF,), _bf16)
      return carry0

    lax.fori_loop(0, _RPT, zb, 0)

    @pl.when(t < NS - 1)
    def _():
      _stage(_RPT)

    @pl.when(t == NS - 1)
    def _():
      _stage(_RLAST)

    plsc.subcore_barrier()

    _idx_start(0, l, 0)
    _idx_start(1, l, 1)
    _idx_wait(0)
    _gather_start(0)

    def pair(j, carry2):
      _idx_wait(1)
      _gather_start(1)
      _gather_wait(0)

      @pl.when(j > 0)
      def _():
        _scatter_wait(0)

      _scale(0)
      _scatter_start(0)
      _idx_start(0, l, 2 * j + 2)
      _idx_wait(0)
      _gather_start(0)
      _gather_wait(1)

      @pl.when(j > 0)
      def _():
        _scatter_wait(1)

      _scale(1)
      _scatter_start(1)

      @pl.when(j < _NPAIRS - 1)
      def _():
        _idx_start(1, l, 2 * j + 3)

      return carry2

    lax.fori_loop(0, _NPAIRS, pair, 0)
    _gather_wait(0)
    _scatter_wait(0)
    _scale(0)
    _scatter_start(0)
    _scatter_wait(0)
    _scatter_wait(1)
    plsc.subcore_barrier()

    @pl.when(t < NS - 1)
    def _():
      _unstage(_RPT, l)

    @pl.when(t == NS - 1)
    def _():
      _unstage(_RLAST, l)

    return carry

  lax.fori_loop(0, L, per_l, 0)


_conv_call = pl.kernel(
    _conv_body,
    out_type=[
        jax.ShapeDtypeStruct((L * N, HALF), _bf16),
        jax.ShapeDtypeStruct((L * N, HALF), _bf16),
    ],
    mesh=plsc.VectorSubcoreMesh(core_axis_name="c", subcore_axis_name="s",
                                num_cores=NC, num_subcores=NS),
    compiler_params=pltpu.CompilerParams(use_tc_tiling_on_sc=False),
    scratch_types=[
        pltpu.VMEM_SHARED((N, HALF), _bf16),
        pltpu.VMEM((1, 2, _CB), _i32),
        pltpu.VMEM((1, 2, _CB), _i32),
        pltpu.VMEM((_CB,), _f32),
        pltpu.VMEM((_CB,), _f32),
        pltpu.VMEM((_CB,), _i32),
        pltpu.VMEM((_CB,), _i32),
        pltpu.VMEM((_CB, HALF), _bf16),
        pltpu.VMEM((_CB, HALF), _bf16),
        pltpu.VMEM((_CB, HALF), _bf16),
        pltpu.VMEM((_CB, HALF), _bf16),
        pltpu.SemaphoreType.DMA,
        pltpu.SemaphoreType.DMA,
        pltpu.SemaphoreType.DMA,
        pltpu.SemaphoreType.DMA,
        pltpu.SemaphoreType.DMA,
        pltpu.SemaphoreType.DMA,
    ],
)

_BN = 2000
_NB = N // _BN


def _prep_block(degp_ref, x_ref, w1_ref, dis_ref, ta_ref, tb_ref,
                tab_ref, tbb_ref):
  deg = degp_ref[0, :, 0] + degp_ref[0, :, 1] + 1.0
  dis = lax.rsqrt(deg)
  dis_ref[0, :, 0] = dis
  tmat = jnp.dot(x_ref[0], w1_ref[...], preferred_element_type=_f32)
  ts = tmat * dis[:, None]
  ta_ref[0] = ts[:, :HALF]
  tb_ref[0] = ts[:, HALF:]
  tab_ref[0] = ts[:, :HALF].astype(jnp.bfloat16)
  tbb_ref[0] = ts[:, HALF:].astype(jnp.bfloat16)


def _prep_call(degp, x, w1):
  return pl.pallas_call(
      _prep_block,
      grid=(L, _NB),
      in_specs=[
          pl.BlockSpec((1, _BN, 2), lambda l, i: (l, i, 0)),
          pl.BlockSpec((1, _BN, D), lambda l, i: (l, i, 0)),
          pl.BlockSpec((D, D), lambda l, i: (0, 0)),
      ],
      out_specs=[
          pl.BlockSpec((1, _BN, 1), lambda l, i: (l, i, 0)),
          pl.BlockSpec((1, _BN, HALF), lambda l, i: (l, i, 0)),
          pl.BlockSpec((1, _BN, HALF), lambda l, i: (l, i, 0)),
          pl.BlockSpec((1, _BN, HALF), lambda l, i: (l, i, 0)),
          pl.BlockSpec((1, _BN, HALF), lambda l, i: (l, i, 0)),
      ],
      out_shape=[
          jax.ShapeDtypeStruct((L, N, 1), _f32),
          jax.ShapeDtypeStruct((L, N, HALF), _f32),
          jax.ShapeDtypeStruct((L, N, HALF), _f32),
          jax.ShapeDtypeStruct((L, N, HALF), jnp.bfloat16),
          jax.ShapeDtypeStruct((L, N, HALF), jnp.bfloat16),
      ],
  )(degp, x, w1)


def _mid_block(sa_ref, sb_ref, ta_ref, tb_ref, dis_ref, b_ref, w_ref,
               ta2_ref, tb2_ref, tab2_ref, tbb2_ref):
  d = dis_ref[0, :, 0]
  s_plus_t = jnp.concatenate(
      [sa_ref[0] + ta_ref[0], sb_ref[0] + tb_ref[0]], axis=1)
  h = jnp.maximum(d[:, None] * s_plus_t + b_ref[0][None, :], 0.0)
  t2 = jnp.dot(h, w_ref[...], preferred_element_type=_f32)
  ts = t2 * d[:, None]
  ta2_ref[0] = ts[:, :HALF]
  tb2_ref[0] = ts[:, HALF:]
  tab2_ref[0] = ts[:, :HALF].astype(jnp.bfloat16)
  tbb2_ref[0] = ts[:, HALF:].astype(jnp.bfloat16)


def _mid_call(sa, sb, ta, tb, dis, b, w):
  return pl.pallas_call(
      _mid_block,
      grid=(L, _NB),
      in_specs=[
          pl.BlockSpec((1, _BN, HALF), lambda l, i: (l, i, 0)),
          pl.BlockSpec((1, _BN, HALF), lambda l, i: (l, i, 0)),
          pl.BlockSpec((1, _BN, HALF), lambda l, i: (l, i, 0)),
          pl.BlockSpec((1, _BN, HALF), lambda l, i: (l, i, 0)),
          pl.BlockSpec((1, _BN, 1), lambda l, i: (l, i, 0)),
          pl.BlockSpec((1, D), lambda l, i: (0, 0)),
          pl.BlockSpec((D, D), lambda l, i: (0, 0)),
      ],
      out_specs=[
          pl.BlockSpec((1, _BN, HALF), lambda l, i: (l, i, 0)),
          pl.BlockSpec((1, _BN, HALF), lambda l, i: (l, i, 0)),
          pl.BlockSpec((1, _BN, HALF), lambda l, i: (l, i, 0)),
          pl.BlockSpec((1, _BN, HALF), lambda l, i: (l, i, 0)),
      ],
      out_shape=[
          jax.ShapeDtypeStruct((L, N, HALF), _f32),
          jax.ShapeDtypeStruct((L, N, HALF), _f32),
          jax.ShapeDtypeStruct((L, N, HALF), jnp.bfloat16),
          jax.ShapeDtypeStruct((L, N, HALF), jnp.bfloat16),
      ],
  )(sa, sb, ta, tb, dis, b, w)


def _fin_block(sa_ref, sb_ref, ta_ref, tb_ref, dis_ref, b_ref, pp_ref):
  d = dis_ref[0, :, 0]
  s_plus_t = jnp.concatenate(
      [sa_ref[0] + ta_ref[0], sb_ref[0] + tb_ref[0]], axis=1)
  z = jnp.maximum(d[:, None] * s_plus_t + b_ref[0][None, :], 0.0)
  pp_ref[0, 0, 0, :] = jnp.sum(z, axis=0)


def _fin_call(sa, sb, ta, tb, dis, b):
  return pl.pallas_call(
      _fin_block,
      grid=(L, _NB),
      in_specs=[
          pl.BlockSpec((1, _BN, HALF), lambda l, i: (l, i, 0)),
          pl.BlockSpec((1, _BN, HALF), lambda l, i: (l, i, 0)),
          pl.BlockSpec((1, _BN, HALF), lambda l, i: (l, i, 0)),
          pl.BlockSpec((1, _BN, HALF), lambda l, i: (l, i, 0)),
          pl.BlockSpec((1, _BN, 1), lambda l, i: (l, i, 0)),
          pl.BlockSpec((1, D), lambda l, i: (0, 0)),
      ],
      out_specs=[pl.BlockSpec((1, 1, 1, D), lambda l, i: (l, i, 0, 0))],
      out_shape=[jax.ShapeDtypeStruct((L, _NB, 1, D), _f32)],
  )(sa, sb, ta, tb, dis, b)[0]


def _rnn_block(pp_ref, wih_t_ref, bih_ref, whh_t_ref, bhh_ref, wout_ref,
               bout_ref, out_ref):
  seq = jnp.sum(pp_ref[...], axis=(1, 2)) * (1.0 / N)
  h = jnp.zeros((1, D), _f32)
  hs = []
  for i in range(L):
    xt = lax.slice(seq, (i, 0), (i + 1, D))
    h = jnp.tanh(
        jnp.dot(xt, wih_t_ref[...], preferred_element_type=_f32)
        + bih_ref[...]
        + jnp.dot(h, whh_t_ref[...], preferred_element_type=_f32)
        + bhh_ref[...])
    hs.append(h)
  hsm = jnp.concatenate(hs, axis=0)
  logits = jnp.dot(hsm, wout_ref[...], preferred_element_type=_f32)
  out_ref[...] = jax.nn.sigmoid(logits + bout_ref[...])


def _rnn_call(pp, wih_t, bih, whh_t, bhh, wout, bout):
  return pl.pallas_call(
      _rnn_block,
      out_shape=jax.ShapeDtypeStruct((L, 2), _f32),
  )(pp, wih_t, bih, whh_t, bhh, wout, bout)


def kernel(x, edge_index, edge_weight, W1, b1, W2, b2, W3, b3,
           W_ih, b_ih, W_hh, b_hh, W_out, b_out):
  loff = (jnp.arange(L, dtype=_i32) * N)[:, None]
  w = edge_weight.reshape(L * E)
  srcr = (edge_index[:, 0, :] + loff).reshape(L, NS, _NCH, _CB)
  dstr = edge_index[:, 1, :].reshape(L, NS, _NCH, _CB)
  epk = jnp.stack([srcr, dstr], axis=3).reshape(L * NS * _NCH, 2, _CB)

  degp = _deg_call(epk, w)
  degt = jnp.transpose(degp.reshape(NC, L, N), (1, 2, 0))

  dis, ta, tb, tab, tbb = _prep_call(degt, x, W1)

  def flat(a):
    return a.reshape(L * N, HALF)

  def unflat(a):
    return a.reshape(L, N, HALF)

  sa, sb = _conv_call(flat(tab), flat(tbb), epk, w)
  ta, tb, tab, tbb = _mid_call(unflat(sa), unflat(sb), ta, tb, dis,
                               b1.reshape(1, D), W2)
  sa, sb = _conv_call(flat(tab), flat(tbb), epk, w)
  ta, tb, tab, tbb = _mid_call(unflat(sa), unflat(sb), ta, tb, dis,
                               b2.reshape(1, D), W3)
  sa, sb = _conv_call(flat(tab), flat(tbb), epk, w)
  pp = _fin_call(unflat(sa), unflat(sb), ta, tb, dis, b3.reshape(1, D))

  return _rnn_call(pp, W_ih.T, b_ih.reshape(1, D), W_hh.T,
                   b_hh.reshape(1, D), W_out, b_out.reshape(1, 2))


# speedup vs baseline: 48.8057x; 1.0093x over previous
import functools

import jax
import jax.numpy as jnp
from jax import lax
from jax.experimental import pallas as pl
from jax.experimental.pallas import tpu as pltpu
from jax.experimental.pallas import tpu_sc as plsc

L, N, D = 8, 10000, 128
E = 320000
HALF = D // 2
NC, NS = 2, 16

_f32 = jnp.float32
_i32 = jnp.int32

_DEG_EPT = E // (NC * NS)
_DEG_CB = 1000
_DEG_NCH = _DEG_EPT // _DEG_CB

_NP = 10240


_CB = 800
_EPT = E // NS
_NCH = _EPT // _CB
_DG = _CB // NC


def _deg_body(epk, w_hbm, out_hbm, acc,
              ebufa, ebufb, wbufa, wbufb, didxa, didxb, wsrca, wsrcb,
              zbuf, tout, isema, isemb, ssema, ssemb):
  c = lax.axis_index("c")
  t = lax.axis_index("s")
  ebuf = (ebufa, ebufb)
  wbuf = (wbufa, wbufb)
  didx = (didxa, didxb)
  wsrc = (wsrca, wsrcb)
  isem = (isema, isemb)
  ssem = (ssema, ssemb)
  for i in range(640 // 16):
    zbuf[pl.ds(i * 16, 16)] = jnp.zeros((16,), _f32)

  def _idx_start(s, l, k):
    row = (l * NS + t) * _NCH + k
    pltpu.async_copy(epk.at[pl.ds(row, 1), :, :], ebuf[s], isem[s])
    eb = pl.multiple_of(l * E + t * _EPT + k * _CB, 8)
    pltpu.async_copy(w_hbm.at[pl.ds(eb, _CB)], wbuf[s], isem[s])

  def _idx_wait(s):
    pltpu.make_async_copy(epk.at[pl.ds(0, 1), :, :], ebuf[s], isem[s]).wait()
    pltpu.make_async_copy(w_hbm.at[pl.ds(0, _CB)], wbuf[s], isem[s]).wait()

  def _scatter_start(s):
    pltpu.async_copy(wsrc[s], acc.at[didx[s]], ssem[s], add=True)

  def _scatter_wait(s):
    pltpu.make_async_copy(wsrc[s], acc.at[didx[s]], ssem[s]).wait()

  def _grab(s):
    def per_g(g, carry3):
      sl = pl.ds(pl.multiple_of(g * 16, 16), 16)
      slh = pl.ds(c * _DG + g * 16, 16)
      didx[s][sl] = ebuf[s][0, 1, slh]
      wsrc[s][sl] = wbuf[s][slh]
      return carry3

    lax.fori_loop(0, _DG // 16, per_g, 0)

  def per_l(l, carry):
    pltpu.sync_copy(zbuf, acc.at[pl.ds(pl.multiple_of(t * 640, 8), 640)])
    plsc.subcore_barrier()

    _idx_start(0, l, 0)
    _idx_start(1, l, 1)

    def pair(j, carry2):
      _idx_wait(0)

      @pl.when(j > 0)
      def _():
        _scatter_wait(0)

      _grab(0)
      _scatter_start(0)
      _idx_start(0, l, 2 * j + 2)
      _idx_wait(1)

      @pl.when(j > 0)
      def _():
        _scatter_wait(1)

      _grab(1)
      _scatter_start(1)

      @pl.when(j < _NPAIRS - 1)
      def _():
        _idx_start(1, l, 2 * j + 3)

      return carry2

    lax.fori_loop(0, _NPAIRS, pair, 0)
    _idx_wait(0)
    _scatter_wait(0)
    _grab(0)
    _scatter_start(0)
    _scatter_wait(0)
    _scatter_wait(1)
    plsc.subcore_barrier()
    ob = pl.multiple_of(c * (L * N) + l * N, 8)

    @pl.when(t < NS - 1)
    def _():
      o = pl.multiple_of(t * 640, 8)
      pltpu.sync_copy(acc.at[pl.ds(o, 640)], tout)
      pltpu.sync_copy(tout, out_hbm.at[pl.ds(ob + o, 640)])

    @pl.when(t == NS - 1)
    def _():
      pltpu.sync_copy(acc.at[pl.ds(9600, 400)], tout.at[pl.ds(0, 400)])
      pltpu.sync_copy(tout.at[pl.ds(0, 400)], out_hbm.at[pl.ds(ob + 9600, 400)])

    return carry

  lax.fori_loop(0, L, per_l, 0)


_deg_call = pl.kernel(
    _deg_body,
    out_type=jax.ShapeDtypeStruct((NC * L * N,), _f32),
    mesh=plsc.VectorSubcoreMesh(core_axis_name="c", subcore_axis_name="s",
                                num_cores=NC, num_subcores=NS),
    compiler_params=pltpu.CompilerParams(use_tc_tiling_on_sc=False),
    scratch_types=[
        pltpu.VMEM_SHARED((_NP,), _f32),
        pltpu.VMEM((1, 2, _CB), _i32),
        pltpu.VMEM((1, 2, _CB), _i32),
        pltpu.VMEM((_CB,), _f32),
        pltpu.VMEM((_CB,), _f32),
        pltpu.VMEM((_DG,), _i32),
        pltpu.VMEM((_DG,), _i32),
        pltpu.VMEM((_DG,), _f32),
        pltpu.VMEM((_DG,), _f32),
        pltpu.VMEM((640,), _f32),
        pltpu.VMEM((640,), _f32),
        pltpu.SemaphoreType.DMA,
        pltpu.SemaphoreType.DMA,
        pltpu.SemaphoreType.DMA,
        pltpu.SemaphoreType.DMA,
    ],
)

_EPT = E // NS
_CB = 800
_NCH = _EPT // _CB
_RPT = 640
_RLAST = N - (NS - 1) * _RPT


_bf16 = jnp.bfloat16
_NPAIRS = _NCH // 2


def _conv_body(tpa, tpb, epk, w_hbm, sa, sb, acc,
               ebufa, ebufb, wbufa, wbufb, didxa, didxb,
               gbufa, gbufb, sbufa, sbufb,
               isema, isemb, gsema, gsemb, ssema, ssemb):
  c = lax.axis_index("c")
  t = lax.axis_index("s")
  ebuf = (ebufa, ebufb)
  wbuf = (wbufa, wbufb)
  didx = (didxa, didxb)
  gbuf = (gbufa, gbufb)
  sbuf = (sbufa, sbufb)
  isem = (isema, isemb)
  gsem = (gsema, gsemb)
  ssem = (ssema, ssemb)

  def _row(l, k):
    return (l * NS + t) * _NCH + k

  def _idx_start(s, l, k):
    pltpu.async_copy(epk.at[pl.ds(_row(l, k), 1), :, :], ebuf[s], isem[s])
    eb = pl.multiple_of(l * E + t * _EPT + k * _CB, 8)
    pltpu.async_copy(w_hbm.at[pl.ds(eb, _CB)], wbuf[s], isem[s])

  def _idx_wait(s):
    pltpu.make_async_copy(epk.at[pl.ds(0, 1), :, :], ebuf[s], isem[s]).wait()
    pltpu.make_async_copy(w_hbm.at[pl.ds(0, _CB)], wbuf[s], isem[s]).wait()

  def _gather_start(s):
    @pl.when(c == 0)
    def _():
      pltpu.async_copy(tpa.at[ebuf[s].at[0, 0]], gbuf[s], gsem[s])

    @pl.when(c == 1)
    def _():
      pltpu.async_copy(tpb.at[ebuf[s].at[0, 0]], gbuf[s], gsem[s])

  def _gather_wait(s):
    @pl.when(c == 0)
    def _():
      pltpu.make_async_copy(tpa.at[ebuf[s].at[0, 0]], gbuf[s], gsem[s]).wait()

    @pl.when(c == 1)
    def _():
      pltpu.make_async_copy(tpb.at[ebuf[s].at[0, 0]], gbuf[s], gsem[s]).wait()

  def _scale(s):
    @plsc.parallel_loop(0, _CB // 16, unroll=4)
    def per_g(g):
      sl = pl.ds(pl.multiple_of(g * 16, 16), 16)
      didx[s][sl] = ebuf[s][0, 1, sl]
      wv = wbuf[s][sl]
      for e16 in range(16):
        e = g * 16 + e16
        ws = jnp.broadcast_to(wv[e16], (HALF,)).astype(_bf16)
        sbuf[s][e, :] = gbuf[s][e, :] * ws

  def _scatter_start(s):
    pltpu.async_copy(sbuf[s], acc.at[didx[s]], ssem[s], add=True)

  def _scatter_wait(s):
    pltpu.make_async_copy(sbuf[s], acc.at[didx[s]], ssem[s]).wait()

  def _stage(nrows):
    ra = pl.multiple_of(t * _RPT, 8)
    pltpu.sync_copy(sbufa.at[pl.ds(0, nrows), :], acc.at[pl.ds(ra, nrows), :])

  def _unstage(nrows, l):
    rb = pl.multiple_of(l * N + t * _RPT, 8)
    ra = pl.multiple_of(t * _RPT, 8)
    pltpu.sync_copy(acc.at[pl.ds(ra, nrows), :], sbufa.at[pl.ds(0, nrows), :])

    @pl.when(c == 0)
    def _():
      pltpu.sync_copy(sbufa.at[pl.ds(0, nrows), :], sa.at[pl.ds(rb, nrows), :])

    @pl.when(c == 1)
    def _():
      pltpu.sync_copy(sbufa.at[pl.ds(0, nrows), :], sb.at[pl.ds(rb, nrows), :])

  def per_l(l, carry):
    def zb(i, carry0):
      sbufa[i, :] = jnp.zeros((HALF,), _bf16)
      return carry0

    lax.fori_loop(0, _RPT, zb, 0)

    @pl.when(t < NS - 1)
    def _():
      _stage(_RPT)

    @pl.when(t == NS - 1)
    def _():
      _stage(_RLAST)

    plsc.subcore_barrier()

    _idx_start(0, l, 0)
    _idx_start(1, l, 1)
    _idx_wait(0)
    _gather_start(0)

    def pair(j, carry2):
      _idx_wait(1)
      _gather_start(1)
      _gather_wait(0)

      @pl.when(j > 0)
      def _():
        _scatter_wait(0)

      _scale(0)
      _scatter_start(0)
      _idx_start(0, l, 2 * j + 2)
      _idx_wait(0)
      _gather_start(0)
      _gather_wait(1)

      @pl.when(j > 0)
      def _():
        _scatter_wait(1)

      _scale(1)
      _scatter_start(1)

      @pl.when(j < _NPAIRS - 1)
      def _():
        _idx_start(1, l, 2 * j + 3)

      return carry2

    lax.fori_loop(0, _NPAIRS, pair, 0)
    _gather_wait(0)
    _scatter_wait(0)
    _scale(0)
    _scatter_start(0)
    _scatter_wait(0)
    _scatter_wait(1)
    plsc.subcore_barrier()

    @pl.when(t < NS - 1)
    def _():
      _unstage(_RPT, l)

    @pl.when(t == NS - 1)
    def _():
      _unstage(_RLAST, l)

    return carry

  lax.fori_loop(0, L, per_l, 0)


_conv_call = pl.kernel(
    _conv_body,
    out_type=[
        jax.ShapeDtypeStruct((L * N, HALF), _bf16),
        jax.ShapeDtypeStruct((L * N, HALF), _bf16),
    ],
    mesh=plsc.VectorSubcoreMesh(core_axis_name="c", subcore_axis_name="s",
                                num_cores=NC, num_subcores=NS),
    compiler_params=pltpu.CompilerParams(use_tc_tiling_on_sc=False),
    scratch_types=[
        pltpu.VMEM_SHARED((N, HALF), _bf16),
        pltpu.VMEM((1, 2, _CB), _i32),
        pltpu.VMEM((1, 2, _CB), _i32),
        pltpu.VMEM((_CB,), _f32),
        pltpu.VMEM((_CB,), _f32),
        pltpu.VMEM((_CB,), _i32),
        pltpu.VMEM((_CB,), _i32),
        pltpu.VMEM((_CB, HALF), _bf16),
        pltpu.VMEM((_CB, HALF), _bf16),
        pltpu.VMEM((_CB, HALF), _bf16),
        pltpu.VMEM((_CB, HALF), _bf16),
        pltpu.SemaphoreType.DMA,
        pltpu.SemaphoreType.DMA,
        pltpu.SemaphoreType.DMA,
        pltpu.SemaphoreType.DMA,
        pltpu.SemaphoreType.DMA,
        pltpu.SemaphoreType.DMA,
    ],
)

_BN = 2000
_NB = N // _BN


def _prep_block(degp_ref, x_ref, w1_ref, dis_ref, tab_ref, tbb_ref):
  deg = degp_ref[0, 0, :, 0] + degp_ref[1, 0, :, 0] + 1.0
  dis = lax.rsqrt(deg)
  dis_ref[0, :, 0] = dis
  tmat = jnp.dot(x_ref[0], w1_ref[...], preferred_element_type=_f32)
  ts = tmat * dis[:, None]
  tab_ref[0] = ts[:, :HALF].astype(jnp.bfloat16)
  tbb_ref[0] = ts[:, HALF:].astype(jnp.bfloat16)


def _prep_call(degp, x, w1):
  return pl.pallas_call(
      _prep_block,
      grid=(L, _NB),
      in_specs=[
          pl.BlockSpec((NC, 1, _BN, 1), lambda l, i: (0, l, i, 0)),
          pl.BlockSpec((1, _BN, D), lambda l, i: (l, i, 0)),
          pl.BlockSpec((D, D), lambda l, i: (0, 0)),
      ],
      out_specs=[
          pl.BlockSpec((1, _BN, 1), lambda l, i: (l, i, 0)),
          pl.BlockSpec((1, _BN, HALF), lambda l, i: (l, i, 0)),
          pl.BlockSpec((1, _BN, HALF), lambda l, i: (l, i, 0)),
      ],
      out_shape=[
          jax.ShapeDtypeStruct((L, N, 1), _f32),
          jax.ShapeDtypeStruct((L, N, HALF), jnp.bfloat16),
          jax.ShapeDtypeStruct((L, N, HALF), jnp.bfloat16),
      ],
  )(degp, x, w1)


def _mid_block(sa_ref, sb_ref, ta_ref, tb_ref, dis_ref, b_ref, w_ref,
               tab2_ref, tbb2_ref):
  d = dis_ref[0, :, 0]
  s_plus_t = jnp.concatenate(
      [sa_ref[0].astype(_f32) + ta_ref[0].astype(_f32),
       sb_ref[0].astype(_f32) + tb_ref[0].astype(_f32)], axis=1)
  h = jnp.maximum(d[:, None] * s_plus_t + b_ref[0][None, :], 0.0)
  t2 = jnp.dot(h, w_ref[...], preferred_element_type=_f32)
  ts = t2 * d[:, None]
  tab2_ref[0] = ts[:, :HALF].astype(jnp.bfloat16)
  tbb2_ref[0] = ts[:, HALF:].astype(jnp.bfloat16)


def _mid_call(sa, sb, ta, tb, dis, b, w):
  return pl.pallas_call(
      _mid_block,
      grid=(L, _NB),
      in_specs=[
          pl.BlockSpec((1, _BN, HALF), lambda l, i: (l, i, 0)),
          pl.BlockSpec((1, _BN, HALF), lambda l, i: (l, i, 0)),
          pl.BlockSpec((1, _BN, HALF), lambda l, i: (l, i, 0)),
          pl.BlockSpec((1, _BN, HALF), lambda l, i: (l, i, 0)),
          pl.BlockSpec((1, _BN, 1), lambda l, i: (l, i, 0)),
          pl.BlockSpec((1, D), lambda l, i: (0, 0)),
          pl.BlockSpec((D, D), lambda l, i: (0, 0)),
      ],
      out_specs=[
          pl.BlockSpec((1, _BN, HALF), lambda l, i: (l, i, 0)),
          pl.BlockSpec((1, _BN, HALF), lambda l, i: (l, i, 0)),
      ],
      out_shape=[
          jax.ShapeDtypeStruct((L, N, HALF), jnp.bfloat16),
          jax.ShapeDtypeStruct((L, N, HALF), jnp.bfloat16),
      ],
  )(sa, sb, ta, tb, dis, b, w)


def _fin_block(sa_ref, sb_ref, ta_ref, tb_ref, dis_ref, b_ref, pp_ref):
  d = dis_ref[0, :, 0]
  s_plus_t = jnp.concatenate(
      [sa_ref[0].astype(_f32) + ta_ref[0].astype(_f32),
       sb_ref[0].astype(_f32) + tb_ref[0].astype(_f32)], axis=1)
  z = jnp.maximum(d[:, None] * s_plus_t + b_ref[0][None, :], 0.0)
  pp_ref[0, 0, 0, :] = jnp.sum(z, axis=0)


def _fin_call(sa, sb, ta, tb, dis, b):
  return pl.pallas_call(
      _fin_block,
      grid=(L, _NB),
      in_specs=[
          pl.BlockSpec((1, _BN, HALF), lambda l, i: (l, i, 0)),
          pl.BlockSpec((1, _BN, HALF), lambda l, i: (l, i, 0)),
          pl.BlockSpec((1, _BN, HALF), lambda l, i: (l, i, 0)),
          pl.BlockSpec((1, _BN, HALF), lambda l, i: (l, i, 0)),
          pl.BlockSpec((1, _BN, 1), lambda l, i: (l, i, 0)),
          pl.BlockSpec((1, D), lambda l, i: (0, 0)),
      ],
      out_specs=[pl.BlockSpec((1, 1, 1, D), lambda l, i: (l, i, 0, 0))],
      out_shape=[jax.ShapeDtypeStruct((L, _NB, 1, D), _f32)],
  )(sa, sb, ta, tb, dis, b)[0]


def _rnn_block(pp_ref, wih_t_ref, bih_ref, whh_t_ref, bhh_ref, wout_ref,
               bout_ref, out_ref):
  seq = jnp.sum(pp_ref[...], axis=(1, 2)) * (1.0 / N)
  h = jnp.zeros((1, D), _f32)
  hs = []
  for i in range(L):
    xt = lax.slice(seq, (i, 0), (i + 1, D))
    h = jnp.tanh(
        jnp.dot(xt, wih_t_ref[...], preferred_element_type=_f32)
        + bih_ref[...]
        + jnp.dot(h, whh_t_ref[...], preferred_element_type=_f32)
        + bhh_ref[...])
    hs.append(h)
  hsm = jnp.concatenate(hs, axis=0)
  logits = jnp.dot(hsm, wout_ref[...], preferred_element_type=_f32)
  out_ref[...] = jax.nn.sigmoid(logits + bout_ref[...])


def _rnn_call(pp, wih_t, bih, whh_t, bhh, wout, bout):
  return pl.pallas_call(
      _rnn_block,
      out_shape=jax.ShapeDtypeStruct((L, 2), _f32),
  )(pp, wih_t, bih, whh_t, bhh, wout, bout)


def kernel(x, edge_index, edge_weight, W1, b1, W2, b2, W3, b3,
           W_ih, b_ih, W_hh, b_hh, W_out, b_out):
  loff = (jnp.arange(L, dtype=_i32) * N)[:, None]
  w = edge_weight.reshape(L * E)
  srcr = (edge_index[:, 0, :] + loff).reshape(L, NS, _NCH, _CB)
  dstr = edge_index[:, 1, :].reshape(L, NS, _NCH, _CB)
  epk = jnp.stack([srcr, dstr], axis=3).reshape(L * NS * _NCH, 2, _CB)

  degp = _deg_call(epk, w)

  dis, tab, tbb = _prep_call(degp.reshape(NC, L, N, 1), x, W1)

  def flat(a):
    return a.reshape(L * N, HALF)

  def unflat(a):
    return a.reshape(L, N, HALF)

  sa, sb = _conv_call(flat(tab), flat(tbb), epk, w)
  tab, tbb = _mid_call(unflat(sa), unflat(sb), tab, tbb, dis,
                       b1.reshape(1, D), W2)
  sa, sb = _conv_call(flat(tab), flat(tbb), epk, w)
  tab, tbb = _mid_call(unflat(sa), unflat(sb), tab, tbb, dis,
                       b2.reshape(1, D), W3)
  sa, sb = _conv_call(flat(tab), flat(tbb), epk, w)
  pp = _fin_call(unflat(sa), unflat(sb), tab, tbb, dis, b3.reshape(1, D))

  return _rnn_call(pp, W_ih.T, b_ih.reshape(1, D), W_hh.T,
                   b_hh.reshape(1, D), W_out, b_out.reshape(1, 2))
